# Initial kernel scaffold; baseline (speedup 1.0000x reference)
import jax
import jax.numpy as jnp
from jax.experimental import pallas as pl


def kernel(x0, x1, degrees, edge_index0, edge_index1, layer_edge_index, Wg0a, bg0a, Wg0b, bg0b, Wg1a, bg1a, Wg1b, bg1b, Wl1, bl1, Wl2, bl2):
    raise NotImplementedError("write your pallas kernel here")


# speedup vs baseline: 17.3514x; 17.3514x over previous
import functools

import jax
import jax.numpy as jnp
from jax import lax
from jax.experimental import pallas as pl
from jax.experimental.pallas import tpu as pltpu
from jax.experimental.pallas import tpu_sc as plsc

N = 10000
E = 320000
NP = 10240
TBL = 2 * NP
NW = 32
CH = 128
K = 8
SUP = CH * K

E_AB = 655360
E_X = 327680
NSUP_AB = E_AB // (NW * SUP)
NSUP_X = E_X // (NW * SUP)

@functools.cache
def _sc_mesh():
    return plsc.VectorSubcoreMesh(core_axis_name="c", subcore_axis_name="s",
                                  num_cores=2, num_subcores=16)


@functools.cache
def _sc_gather_scatter(nsup):
    rps = TBL // 16

    @functools.partial(
        pl.kernel,
        out_type=jax.ShapeDtypeStruct((2, TBL, 32), jnp.float32),
        mesh=_sc_mesh(),
        compiler_params=pltpu.CompilerParams(use_tc_tiling_on_sc=False),
        scratch_types=[
            pltpu.VMEM((K, CH), jnp.int32),
            pltpu.VMEM((K, CH), jnp.int32),
            pltpu.VMEM((K, CH, 32), jnp.float32),
            pltpu.VMEM_SHARED((TBL, 32), jnp.float32),
            pltpu.SemaphoreType.DMA,
            pltpu.SemaphoreType.DMA,
        ],
    )
    def kern(table_h, src_h, dst_h, zeros_h, out_h,
             idx_s, idx_d, rows, acc, gsem, ssem):
        c = lax.axis_index("c")
        s = lax.axis_index("s")
        wid = s * 2 + c
        pltpu.sync_copy(zeros_h.at[pl.ds(s * rps, rps)],
                        acc.at[pl.ds(s * rps, rps)])
        plsc.subcore_barrier()

        @pl.loop(0, nsup)
        def _(j):
            ch0 = (wid * nsup + j) * K
            pltpu.sync_copy(src_h.at[pl.ds(ch0, K)], idx_s)
            pltpu.sync_copy(dst_h.at[pl.ds(ch0, K)], idx_d)
            gets = [pltpu.async_copy(table_h.at[idx_s.at[t]], rows.at[t], gsem)
                    for t in range(K)]
            for cp in gets:
                cp.wait()
            puts = [pltpu.async_copy(rows.at[t], acc.at[idx_d.at[t]], ssem,
                                     add=True)
                    for t in range(K)]
            for cp in puts:
                cp.wait()

        plsc.subcore_barrier()
        pltpu.sync_copy(acc.at[pl.ds(s * rps, rps)],
                        out_h.at[c].at[pl.ds(s * rps, rps)])

    return kern


@functools.cache
def _sc_degree(nsup):
    rps = TBL // 16

    @functools.partial(
        pl.kernel,
        out_type=jax.ShapeDtypeStruct((2, TBL, 16), jnp.float32),
        mesh=_sc_mesh(),
        compiler_params=pltpu.CompilerParams(use_tc_tiling_on_sc=False),
        scratch_types=[
            pltpu.VMEM((K, CH), jnp.int32),
            pltpu.VMEM((CH, 16), jnp.float32),
            pltpu.VMEM_SHARED((TBL, 16), jnp.float32),
            pltpu.SemaphoreType.DMA,
        ],
    )
    def kern(dst_h, zeros_h, ones_h, out_h, idx_d, ones_v, acc, ssem):
        c = lax.axis_index("c")
        s = lax.axis_index("s")
        wid = s * 2 + c
        pltpu.sync_copy(zeros_h.at[pl.ds(s * rps, rps)],
                        acc.at[pl.ds(s * rps, rps)])
        pltpu.sync_copy(ones_h, ones_v)
        plsc.subcore_barrier()

        @pl.loop(0, nsup)
        def _(j):
            ch0 = (wid * nsup + j) * K
            pltpu.sync_copy(dst_h.at[pl.ds(ch0, K)], idx_d)
            puts = [pltpu.async_copy(ones_v, acc.at[idx_d.at[t]], ssem,
                                     add=True)
                    for t in range(K)]
            for cp in puts:
                cp.wait()

        plsc.subcore_barrier()
        pltpu.sync_copy(acc.at[pl.ds(s * rps, rps)],
                        out_h.at[c].at[pl.ds(s * rps, rps)])

    return kern


RB1 = 512
RB = 1024


def _tc1_body(x0_ref, x1_ref, w0_ref, w1_ref, dp_ref, z_ref, di_ref):
    d0 = dp_ref[0, 0] + dp_ref[1, 0]
    d1 = dp_ref[0, 1] + dp_ref[1, 1]
    di0 = lax.rsqrt(d0[:, 0:1] + 1.0)
    di1 = lax.rsqrt(d1[:, 0:1] + 1.0)
    z_ref[0] = jnp.dot(x0_ref[...], w0_ref[...],
                       preferred_element_type=jnp.float32) * di0
    z_ref[1] = jnp.dot(x1_ref[...], w1_ref[...],
                       preferred_element_type=jnp.float32) * di1
    di_ref[0] = jnp.broadcast_to(di0, (RB1, 32))
    di_ref[1] = jnp.broadcast_to(di1, (RB1, 32))


def _tc1(x0p, x1p, w0, w1, degp):
    return pl.pallas_call(
        _tc1_body,
        grid=(NP // RB1,),
        in_specs=[
            pl.BlockSpec((RB1, 128), lambda i: (i, 0)),
            pl.BlockSpec((RB1, 128), lambda i: (i, 0)),
            pl.BlockSpec((128, 32), lambda i: (0, 0)),
            pl.BlockSpec((128, 32), lambda i: (0, 0)),
            pl.BlockSpec((2, 2, RB1, 16), lambda i: (0, 0, i, 0)),
        ],
        out_specs=[
            pl.BlockSpec((2, RB1, 32), lambda i: (0, i, 0)),
            pl.BlockSpec((2, RB1, 32), lambda i: (0, i, 0)),
        ],
        out_shape=[
            jax.ShapeDtypeStruct((2, NP, 32), jnp.float32),
            jax.ShapeDtypeStruct((2, NP, 32), jnp.float32),
        ],
    )(x0p, x1p, w0, w1, degp)


def _tc2_body(pa_ref, z_ref, di_ref, w0_ref, w1_ref, b0_ref, b1_ref, zb_ref):
    a0 = pa_ref[0, 0] + pa_ref[1, 0] + z_ref[0]
    a1 = pa_ref[0, 1] + pa_ref[1, 1] + z_ref[1]
    h0 = jnp.maximum(di_ref[0] * a0 + b0_ref[...], 0.0)
    h1 = jnp.maximum(di_ref[1] * a1 + b1_ref[...], 0.0)
    zb_ref[0] = jnp.dot(h0, w0_ref[...],
                        preferred_element_type=jnp.float32) * di_ref[0]
    zb_ref[1] = jnp.dot(h1, w1_ref[...],
                        preferred_element_type=jnp.float32) * di_ref[1]


def _tc2(pa, z, di, w0b, w1b, b0a, b1a):
    return pl.pallas_call(
        _tc2_body,
        grid=(NP // RB,),
        in_specs=[
            pl.BlockSpec((2, 2, RB, 32), lambda i: (0, 0, i, 0)),
            pl.BlockSpec((2, RB, 32), lambda i: (0, i, 0)),
            pl.BlockSpec((2, RB, 32), lambda i: (0, i, 0)),
            pl.BlockSpec((32, 32), lambda i: (0, 0)),
            pl.BlockSpec((32, 32), lambda i: (0, 0)),
            pl.BlockSpec((1, 32), lambda i: (0, 0)),
            pl.BlockSpec((1, 32), lambda i: (0, 0)),
        ],
        out_specs=pl.BlockSpec((2, RB, 32), lambda i: (0, i, 0)),
        out_shape=jax.ShapeDtypeStruct((2, NP, 32), jnp.float32),
    )(pa, z, di, w0b, w1b, b0a, b1a)


def _tc3_body(pb_ref, zb_ref, di_ref, b0_ref, b1_ref, f_ref):
    f_ref[0] = di_ref[0] * (pb_ref[0, 0] + pb_ref[1, 0] + zb_ref[0]) + b0_ref[...]
    f_ref[1] = di_ref[1] * (pb_ref[0, 1] + pb_ref[1, 1] + zb_ref[1]) + b1_ref[...]


def _tc3(pb, zb, di, b0b, b1b):
    return pl.pallas_call(
        _tc3_body,
        grid=(NP // RB,),
        in_specs=[
            pl.BlockSpec((2, 2, RB, 32), lambda i: (0, 0, i, 0)),
            pl.BlockSpec((2, RB, 32), lambda i: (0, i, 0)),
            pl.BlockSpec((2, RB, 32), lambda i: (0, i, 0)),
            pl.BlockSpec((1, 32), lambda i: (0, 0)),
            pl.BlockSpec((1, 32), lambda i: (0, 0)),
        ],
        out_specs=pl.BlockSpec((2, RB, 32), lambda i: (0, i, 0)),
        out_shape=jax.ShapeDtypeStruct((2, NP, 32), jnp.float32),
    )(pb, zb, di, b0b, b1b)


def _tc4_body(f_ref, pc_ref, invd_ref, wl1_ref, bl1_ref, wl2_ref, bl2_ref,
              o_ref):
    last = (f_ref[0] + pc_ref[0, 0] + pc_ref[1, 0]) * invd_ref[...]
    h = jnp.maximum(jnp.dot(last, wl1_ref[...],
                            preferred_element_type=jnp.float32)
                    + bl1_ref[...], 0.0)
    o_ref[...] = jnp.maximum(jnp.dot(h, wl2_ref[...],
                                     preferred_element_type=jnp.float32)
                             + bl2_ref[...], 0.0)


def _tc4(f, pc, invd, wl1, bl1, wl2, bl2):
    return pl.pallas_call(
        _tc4_body,
        grid=(NP // RB,),
        in_specs=[
            pl.BlockSpec((1, RB, 32), lambda i: (1, i, 0)),
            pl.BlockSpec((2, 1, RB, 32), lambda i: (0, 0, i, 0)),
            pl.BlockSpec((1, 32), lambda i: (0, 0)),
            pl.BlockSpec((32, 64), lambda i: (0, 0)),
            pl.BlockSpec((1, 64), lambda i: (0, 0)),
            pl.BlockSpec((64, 32), lambda i: (0, 0)),
            pl.BlockSpec((1, 32), lambda i: (0, 0)),
        ],
        out_specs=pl.BlockSpec((RB, 32), lambda i: (i, 0)),
        out_shape=jax.ShapeDtypeStruct((N, 32), jnp.float32),
    )(f, pc, invd, wl1, bl1, wl2, bl2)


def kernel(x0, x1, degrees, edge_index0, edge_index1, layer_edge_index,
           Wg0a, bg0a, Wg0b, bg0b, Wg1a, bg1a, Wg1b, bg1b,
           Wl1, bl1, Wl2, bl2):
    i32 = jnp.int32
    pad_ab = jnp.full((E_AB - 2 * E,), TBL - 1, i32)
    src_ab = jnp.concatenate(
        [edge_index0[0], edge_index1[0] + NP, pad_ab]).reshape(-1, CH)
    dst_ab = jnp.concatenate(
        [edge_index0[1], edge_index1[1] + NP, pad_ab]).reshape(-1, CH)
    pad_x = jnp.full((E_X - E,), TBL - 1, i32)
    src_x = jnp.concatenate([layer_edge_index[1], pad_x]).reshape(-1, CH)
    dst_x = jnp.concatenate([layer_edge_index[0], pad_x]).reshape(-1, CH)

    x0p = jnp.pad(x0, ((0, NP - N), (0, 0)))
    x1p = jnp.pad(x1, ((0, NP - N), (0, 0)))
    zeros32 = jnp.zeros((TBL, 32), jnp.float32)
    zeros16 = jnp.zeros((TBL, 16), jnp.float32)
    ones16 = jnp.ones((CH, 16), jnp.float32)
    invd = (1.0 / degrees[1]) * jnp.ones((1, 32), jnp.float32)

    degp = _sc_degree(NSUP_AB)(dst_ab, zeros16, ones16).reshape(2, 2, NP, 16)
    z, di = _tc1(x0p, x1p, Wg0a, Wg1a, degp)
    pa = _sc_gather_scatter(NSUP_AB)(z.reshape(TBL, 32), src_ab, dst_ab,
                                     zeros32)
    zb = _tc2(pa.reshape(2, 2, NP, 32), z, di, Wg0b, Wg1b,
              bg0a.reshape(1, 32), bg1a.reshape(1, 32))
    pb = _sc_gather_scatter(NSUP_AB)(zb.reshape(TBL, 32), src_ab, dst_ab,
                                     zeros32)
    f = _tc3(pb.reshape(2, 2, NP, 32), zb,
             di, bg0b.reshape(1, 32), bg1b.reshape(1, 32))
    pc = _sc_gather_scatter(NSUP_X)(f.reshape(TBL, 32), src_x, dst_x, zeros32)
    out = _tc4(f, pc.reshape(2, 2, NP, 32), invd,
               Wl1, bl1.reshape(1, 64), Wl2, bl2.reshape(1, 32))
    return out


# speedup vs baseline: 17.8940x; 1.0313x over previous
import functools

import jax
import jax.numpy as jnp
from jax import lax
from jax.experimental import pallas as pl
from jax.experimental.pallas import tpu as pltpu
from jax.experimental.pallas import tpu_sc as plsc

N = 10000
E = 320000
NP = 10240
TBL = 2 * NP
NW = 32
CH = 128
K = 16
SUP = CH * K

E_AB = 655360
E_X = 327680
NSUP_AB = E_AB // (NW * SUP)
NSUP_X = E_X // (NW * SUP)

@functools.cache
def _sc_mesh():
    return plsc.VectorSubcoreMesh(core_axis_name="c", subcore_axis_name="s",
                                  num_cores=2, num_subcores=16)


@functools.cache
def _sc_gather_scatter(nsup):
    rps = TBL // 16

    @functools.partial(
        pl.kernel,
        out_type=jax.ShapeDtypeStruct((2, TBL, 32), jnp.float32),
        mesh=_sc_mesh(),
        compiler_params=pltpu.CompilerParams(use_tc_tiling_on_sc=False),
        scratch_types=[
            pltpu.VMEM((K, CH), jnp.int32),
            pltpu.VMEM((K, CH), jnp.int32),
            pltpu.VMEM((K, CH, 32), jnp.float32),
            pltpu.VMEM_SHARED((TBL, 32), jnp.float32),
            pltpu.SemaphoreType.DMA,
            pltpu.SemaphoreType.DMA,
        ],
    )
    def kern(table_h, src_h, dst_h, zeros_h, out_h,
             idx_s, idx_d, rows, acc, gsem, ssem):
        c = lax.axis_index("c")
        s = lax.axis_index("s")
        wid = s * 2 + c
        pltpu.sync_copy(zeros_h.at[pl.ds(s * rps, rps)],
                        acc.at[pl.ds(s * rps, rps)])
        plsc.subcore_barrier()

        @pl.loop(0, nsup)
        def _(j):
            ch0 = (wid * nsup + j) * K
            pltpu.sync_copy(src_h.at[pl.ds(ch0, K)], idx_s)
            pltpu.sync_copy(dst_h.at[pl.ds(ch0, K)], idx_d)
            gets = [pltpu.async_copy(table_h.at[idx_s.at[t]], rows.at[t], gsem)
                    for t in range(K)]
            for cp in gets:
                cp.wait()
            puts = [pltpu.async_copy(rows.at[t], acc.at[idx_d.at[t]], ssem,
                                     add=True)
                    for t in range(K)]
            for cp in puts:
                cp.wait()

        plsc.subcore_barrier()
        pltpu.sync_copy(acc.at[pl.ds(s * rps, rps)],
                        out_h.at[c].at[pl.ds(s * rps, rps)])

    return kern


@functools.cache
def _sc_degree(nsup):
    rps = TBL // 16

    @functools.partial(
        pl.kernel,
        out_type=jax.ShapeDtypeStruct((2, TBL, 16), jnp.float32),
        mesh=_sc_mesh(),
        compiler_params=pltpu.CompilerParams(use_tc_tiling_on_sc=False),
        scratch_types=[
            pltpu.VMEM((K, CH), jnp.int32),
            pltpu.VMEM((CH, 16), jnp.float32),
            pltpu.VMEM_SHARED((TBL, 16), jnp.float32),
            pltpu.SemaphoreType.DMA,
        ],
    )
    def kern(dst_h, zeros_h, ones_h, out_h, idx_d, ones_v, acc, ssem):
        c = lax.axis_index("c")
        s = lax.axis_index("s")
        wid = s * 2 + c
        pltpu.sync_copy(zeros_h.at[pl.ds(s * rps, rps)],
                        acc.at[pl.ds(s * rps, rps)])
        pltpu.sync_copy(ones_h, ones_v)
        plsc.subcore_barrier()

        @pl.loop(0, nsup)
        def _(j):
            ch0 = (wid * nsup + j) * K
            pltpu.sync_copy(dst_h.at[pl.ds(ch0, K)], idx_d)
            puts = [pltpu.async_copy(ones_v, acc.at[idx_d.at[t]], ssem,
                                     add=True)
                    for t in range(K)]
            for cp in puts:
                cp.wait()

        plsc.subcore_barrier()
        pltpu.sync_copy(acc.at[pl.ds(s * rps, rps)],
                        out_h.at[c].at[pl.ds(s * rps, rps)])

    return kern


RB1 = 512
RB = 1024


def _tc1_body(x0_ref, x1_ref, w0_ref, w1_ref, dp_ref, z_ref, di_ref):
    d0 = dp_ref[0, 0] + dp_ref[1, 0]
    d1 = dp_ref[0, 1] + dp_ref[1, 1]
    di0 = lax.rsqrt(d0[:, 0:1] + 1.0)
    di1 = lax.rsqrt(d1[:, 0:1] + 1.0)
    z_ref[0] = jnp.dot(x0_ref[...], w0_ref[...],
                       preferred_element_type=jnp.float32) * di0
    z_ref[1] = jnp.dot(x1_ref[...], w1_ref[...],
                       preferred_element_type=jnp.float32) * di1
    di_ref[0] = jnp.broadcast_to(di0, (RB1, 32))
    di_ref[1] = jnp.broadcast_to(di1, (RB1, 32))


def _tc1(x0p, x1p, w0, w1, degp):
    return pl.pallas_call(
        _tc1_body,
        grid=(NP // RB1,),
        in_specs=[
            pl.BlockSpec((RB1, 128), lambda i: (i, 0)),
            pl.BlockSpec((RB1, 128), lambda i: (i, 0)),
            pl.BlockSpec((128, 32), lambda i: (0, 0)),
            pl.BlockSpec((128, 32), lambda i: (0, 0)),
            pl.BlockSpec((2, 2, RB1, 16), lambda i: (0, 0, i, 0)),
        ],
        out_specs=[
            pl.BlockSpec((2, RB1, 32), lambda i: (0, i, 0)),
            pl.BlockSpec((2, RB1, 32), lambda i: (0, i, 0)),
        ],
        out_shape=[
            jax.ShapeDtypeStruct((2, NP, 32), jnp.float32),
            jax.ShapeDtypeStruct((2, NP, 32), jnp.float32),
        ],
    )(x0p, x1p, w0, w1, degp)


def _tc2_body(pa_ref, z_ref, di_ref, w0_ref, w1_ref, b0_ref, b1_ref, zb_ref):
    a0 = pa_ref[0, 0] + pa_ref[1, 0] + z_ref[0]
    a1 = pa_ref[0, 1] + pa_ref[1, 1] + z_ref[1]
    h0 = jnp.maximum(di_ref[0] * a0 + b0_ref[...], 0.0)
    h1 = jnp.maximum(di_ref[1] * a1 + b1_ref[...], 0.0)
    zb_ref[0] = jnp.dot(h0, w0_ref[...],
                        preferred_element_type=jnp.float32) * di_ref[0]
    zb_ref[1] = jnp.dot(h1, w1_ref[...],
                        preferred_element_type=jnp.float32) * di_ref[1]


def _tc2(pa, z, di, w0b, w1b, b0a, b1a):
    return pl.pallas_call(
        _tc2_body,
        grid=(NP // RB,),
        in_specs=[
            pl.BlockSpec((2, 2, RB, 32), lambda i: (0, 0, i, 0)),
            pl.BlockSpec((2, RB, 32), lambda i: (0, i, 0)),
            pl.BlockSpec((2, RB, 32), lambda i: (0, i, 0)),
            pl.BlockSpec((32, 32), lambda i: (0, 0)),
            pl.BlockSpec((32, 32), lambda i: (0, 0)),
            pl.BlockSpec((1, 32), lambda i: (0, 0)),
            pl.BlockSpec((1, 32), lambda i: (0, 0)),
        ],
        out_specs=pl.BlockSpec((2, RB, 32), lambda i: (0, i, 0)),
        out_shape=jax.ShapeDtypeStruct((2, NP, 32), jnp.float32),
    )(pa, z, di, w0b, w1b, b0a, b1a)


def _tc3_body(pb_ref, zb_ref, di_ref, b0_ref, b1_ref, f_ref):
    f_ref[0] = di_ref[0] * (pb_ref[0, 0] + pb_ref[1, 0] + zb_ref[0]) + b0_ref[...]
    f_ref[1] = di_ref[1] * (pb_ref[0, 1] + pb_ref[1, 1] + zb_ref[1]) + b1_ref[...]


def _tc3(pb, zb, di, b0b, b1b):
    return pl.pallas_call(
        _tc3_body,
        grid=(NP // RB,),
        in_specs=[
            pl.BlockSpec((2, 2, RB, 32), lambda i: (0, 0, i, 0)),
            pl.BlockSpec((2, RB, 32), lambda i: (0, i, 0)),
            pl.BlockSpec((2, RB, 32), lambda i: (0, i, 0)),
            pl.BlockSpec((1, 32), lambda i: (0, 0)),
            pl.BlockSpec((1, 32), lambda i: (0, 0)),
        ],
        out_specs=pl.BlockSpec((2, RB, 32), lambda i: (0, i, 0)),
        out_shape=jax.ShapeDtypeStruct((2, NP, 32), jnp.float32),
    )(pb, zb, di, b0b, b1b)


def _tc4_body(f_ref, pc_ref, invd_ref, wl1_ref, bl1_ref, wl2_ref, bl2_ref,
              o_ref):
    last = (f_ref[0] + pc_ref[0, 0] + pc_ref[1, 0]) * invd_ref[...]
    h = jnp.maximum(jnp.dot(last, wl1_ref[...],
                            preferred_element_type=jnp.float32)
                    + bl1_ref[...], 0.0)
    o_ref[...] = jnp.maximum(jnp.dot(h, wl2_ref[...],
                                     preferred_element_type=jnp.float32)
                             + bl2_ref[...], 0.0)


def _tc4(f, pc, invd, wl1, bl1, wl2, bl2):
    return pl.pallas_call(
        _tc4_body,
        grid=(NP // RB,),
        in_specs=[
            pl.BlockSpec((1, RB, 32), lambda i: (1, i, 0)),
            pl.BlockSpec((2, 1, RB, 32), lambda i: (0, 0, i, 0)),
            pl.BlockSpec((1, 32), lambda i: (0, 0)),
            pl.BlockSpec((32, 64), lambda i: (0, 0)),
            pl.BlockSpec((1, 64), lambda i: (0, 0)),
            pl.BlockSpec((64, 32), lambda i: (0, 0)),
            pl.BlockSpec((1, 32), lambda i: (0, 0)),
        ],
        out_specs=pl.BlockSpec((RB, 32), lambda i: (i, 0)),
        out_shape=jax.ShapeDtypeStruct((N, 32), jnp.float32),
    )(f, pc, invd, wl1, bl1, wl2, bl2)


def kernel(x0, x1, degrees, edge_index0, edge_index1, layer_edge_index,
           Wg0a, bg0a, Wg0b, bg0b, Wg1a, bg1a, Wg1b, bg1b,
           Wl1, bl1, Wl2, bl2):
    i32 = jnp.int32
    pad_ab = jnp.full((E_AB - 2 * E,), TBL - 1, i32)
    src_ab = jnp.concatenate(
        [edge_index0[0], edge_index1[0] + NP, pad_ab]).reshape(-1, CH)
    dst_ab = jnp.concatenate(
        [edge_index0[1], edge_index1[1] + NP, pad_ab]).reshape(-1, CH)
    pad_x = jnp.full((E_X - E,), TBL - 1, i32)
    src_x = jnp.concatenate([layer_edge_index[1], pad_x]).reshape(-1, CH)
    dst_x = jnp.concatenate([layer_edge_index[0], pad_x]).reshape(-1, CH)

    x0p = jnp.pad(x0, ((0, NP - N), (0, 0)))
    x1p = jnp.pad(x1, ((0, NP - N), (0, 0)))
    zeros32 = jnp.zeros((TBL, 32), jnp.float32)
    zeros16 = jnp.zeros((TBL, 16), jnp.float32)
    ones16 = jnp.ones((CH, 16), jnp.float32)
    invd = (1.0 / degrees[1]) * jnp.ones((1, 32), jnp.float32)

    degp = _sc_degree(NSUP_AB)(dst_ab, zeros16, ones16).reshape(2, 2, NP, 16)
    z, di = _tc1(x0p, x1p, Wg0a, Wg1a, degp)
    pa = _sc_gather_scatter(NSUP_AB)(z.reshape(TBL, 32), src_ab, dst_ab,
                                     zeros32)
    zb = _tc2(pa.reshape(2, 2, NP, 32), z, di, Wg0b, Wg1b,
              bg0a.reshape(1, 32), bg1a.reshape(1, 32))
    pb = _sc_gather_scatter(NSUP_AB)(zb.reshape(TBL, 32), src_ab, dst_ab,
                                     zeros32)
    f = _tc3(pb.reshape(2, 2, NP, 32), zb,
             di, bg0b.reshape(1, 32), bg1b.reshape(1, 32))
    pc = _sc_gather_scatter(NSUP_X)(f.reshape(TBL, 32), src_x, dst_x, zeros32)
    out = _tc4(f, pc.reshape(2, 2, NP, 32), invd,
               Wl1, bl1.reshape(1, 64), Wl2, bl2.reshape(1, 32))
    return out


# speedup vs baseline: 35.3807x; 1.9772x over previous
import functools

import jax
import jax.numpy as jnp
from jax import lax
from jax.experimental import pallas as pl
from jax.experimental.pallas import tpu as pltpu
from jax.experimental.pallas import tpu_sc as plsc

N = 10000
E = 320000
NP = 10240
TBL = 2 * NP
NW = 32
CH = 128
K = 16
SUP = CH * K

E_AB = 655360
E_X = 327680
NSUP_AB = E_AB // (NW * SUP)
NSUP_X = E_X // (NW * SUP)

@functools.cache
def _sc_mesh():
    return plsc.VectorSubcoreMesh(core_axis_name="c", subcore_axis_name="s",
                                  num_cores=2, num_subcores=16)


@functools.cache
def _sc_gather_scatter(nsup):
    rps = TBL // 16

    @functools.partial(
        pl.kernel,
        out_type=jax.ShapeDtypeStruct((2, TBL, 32), jnp.float32),
        mesh=_sc_mesh(),
        compiler_params=pltpu.CompilerParams(use_tc_tiling_on_sc=False),
        scratch_types=[
            pltpu.VMEM((K, CH), jnp.int32),
            pltpu.VMEM((K, CH), jnp.int32),
            pltpu.VMEM((K, CH, 32), jnp.float32),
            pltpu.VMEM_SHARED((TBL, 32), jnp.float32),
            pltpu.SemaphoreType.DMA,
            pltpu.SemaphoreType.DMA,
        ],
    )
    def kern(table_h, src_h, dst_h, zeros_h, out_h,
             idx_s, idx_d, rows, acc, gsem, ssem):
        c = lax.axis_index("c")
        s = lax.axis_index("s")
        wid = s * 2 + c
        pltpu.sync_copy(zeros_h.at[pl.ds(s * rps, rps)],
                        acc.at[pl.ds(s * rps, rps)])
        plsc.subcore_barrier()

        @pl.loop(0, nsup)
        def _(j):
            ch0 = (wid * nsup + j) * K
            pltpu.sync_copy(src_h.at[pl.ds(ch0, K)], idx_s)
            pltpu.sync_copy(dst_h.at[pl.ds(ch0, K)], idx_d)
            gets = [pltpu.async_copy(table_h.at[idx_s.at[t]], rows.at[t], gsem)
                    for t in range(K)]
            for cp in gets:
                cp.wait()
            puts = [pltpu.async_copy(rows.at[t], acc.at[idx_d.at[t]], ssem,
                                     add=True)
                    for t in range(K)]
            for cp in puts:
                cp.wait()

        plsc.subcore_barrier()
        pltpu.sync_copy(acc.at[pl.ds(s * rps, rps)],
                        out_h.at[c].at[pl.ds(s * rps, rps)])

    return kern


@functools.cache
def _sc_degree(nsup):
    rps = TBL // 16

    @functools.partial(
        pl.kernel,
        out_type=jax.ShapeDtypeStruct((2, TBL, 16), jnp.float32),
        mesh=_sc_mesh(),
        compiler_params=pltpu.CompilerParams(use_tc_tiling_on_sc=False),
        scratch_types=[
            pltpu.VMEM((K, CH), jnp.int32),
            pltpu.VMEM((CH, 16), jnp.float32),
            pltpu.VMEM_SHARED((TBL, 16), jnp.float32),
            pltpu.SemaphoreType.DMA,
        ],
    )
    def kern(dst_h, zeros_h, ones_h, out_h, idx_d, ones_v, acc, ssem):
        c = lax.axis_index("c")
        s = lax.axis_index("s")
        wid = s * 2 + c
        pltpu.sync_copy(zeros_h.at[pl.ds(s * rps, rps)],
                        acc.at[pl.ds(s * rps, rps)])
        pltpu.sync_copy(ones_h, ones_v)
        plsc.subcore_barrier()

        @pl.loop(0, nsup)
        def _(j):
            ch0 = (wid * nsup + j) * K
            pltpu.sync_copy(dst_h.at[pl.ds(ch0, K)], idx_d)
            puts = [pltpu.async_copy(ones_v, acc.at[idx_d.at[t]], ssem,
                                     add=True)
                    for t in range(K)]
            for cp in puts:
                cp.wait()

        plsc.subcore_barrier()
        pltpu.sync_copy(acc.at[pl.ds(s * rps, rps)],
                        out_h.at[c].at[pl.ds(s * rps, rps)])

    return kern


RB1 = 512
RB = 1024


def _tc1_body(x0_ref, x1_ref, w0_ref, w1_ref, dp_ref, z_ref, di_ref):
    d0 = dp_ref[0, 0] + dp_ref[1, 0]
    d1 = dp_ref[0, 1] + dp_ref[1, 1]
    di0 = lax.rsqrt(d0[:, 0:1] + 1.0)
    di1 = lax.rsqrt(d1[:, 0:1] + 1.0)
    z_ref[0] = jnp.dot(x0_ref[...], w0_ref[...],
                       preferred_element_type=jnp.float32) * di0
    z_ref[1] = jnp.dot(x1_ref[...], w1_ref[...],
                       preferred_element_type=jnp.float32) * di1
    di_ref[0] = jnp.broadcast_to(di0, (RB1, 32))
    di_ref[1] = jnp.broadcast_to(di1, (RB1, 32))


def _tc1(x0p, x1p, w0, w1, degp):
    return pl.pallas_call(
        _tc1_body,
        grid=(NP // RB1,),
        in_specs=[
            pl.BlockSpec((RB1, 128), lambda i: (i, 0)),
            pl.BlockSpec((RB1, 128), lambda i: (i, 0)),
            pl.BlockSpec((128, 32), lambda i: (0, 0)),
            pl.BlockSpec((128, 32), lambda i: (0, 0)),
            pl.BlockSpec((2, 2, RB1, 16), lambda i: (0, 0, i, 0)),
        ],
        out_specs=[
            pl.BlockSpec((2, RB1, 32), lambda i: (0, i, 0)),
            pl.BlockSpec((2, RB1, 32), lambda i: (0, i, 0)),
        ],
        out_shape=[
            jax.ShapeDtypeStruct((2, NP, 32), jnp.float32),
            jax.ShapeDtypeStruct((2, NP, 32), jnp.float32),
        ],
    )(x0p, x1p, w0, w1, degp)


def _tc2_body(pa_ref, z_ref, di_ref, w0_ref, w1_ref, b0_ref, b1_ref, zb_ref):
    a0 = pa_ref[0, 0] + pa_ref[1, 0] + z_ref[0]
    a1 = pa_ref[0, 1] + pa_ref[1, 1] + z_ref[1]
    h0 = jnp.maximum(di_ref[0] * a0 + b0_ref[...], 0.0)
    h1 = jnp.maximum(di_ref[1] * a1 + b1_ref[...], 0.0)
    zb_ref[0] = jnp.dot(h0, w0_ref[...],
                        preferred_element_type=jnp.float32) * di_ref[0]
    zb_ref[1] = jnp.dot(h1, w1_ref[...],
                        preferred_element_type=jnp.float32) * di_ref[1]


def _tc2(pa, z, di, w0b, w1b, b0a, b1a):
    return pl.pallas_call(
        _tc2_body,
        grid=(NP // RB,),
        in_specs=[
            pl.BlockSpec((2, 2, RB, 32), lambda i: (0, 0, i, 0)),
            pl.BlockSpec((2, RB, 32), lambda i: (0, i, 0)),
            pl.BlockSpec((2, RB, 32), lambda i: (0, i, 0)),
            pl.BlockSpec((32, 32), lambda i: (0, 0)),
            pl.BlockSpec((32, 32), lambda i: (0, 0)),
            pl.BlockSpec((1, 32), lambda i: (0, 0)),
            pl.BlockSpec((1, 32), lambda i: (0, 0)),
        ],
        out_specs=pl.BlockSpec((2, RB, 32), lambda i: (0, i, 0)),
        out_shape=jax.ShapeDtypeStruct((2, NP, 32), jnp.float32),
    )(pa, z, di, w0b, w1b, b0a, b1a)


def _tc3_body(pb_ref, zb_ref, di_ref, b0_ref, b1_ref, f_ref):
    f_ref[0] = di_ref[0] * (pb_ref[0, 0] + pb_ref[1, 0] + zb_ref[0]) + b0_ref[...]
    f_ref[1] = di_ref[1] * (pb_ref[0, 1] + pb_ref[1, 1] + zb_ref[1]) + b1_ref[...]


def _tc3(pb, zb, di, b0b, b1b):
    return pl.pallas_call(
        _tc3_body,
        grid=(NP // RB,),
        in_specs=[
            pl.BlockSpec((2, 2, RB, 32), lambda i: (0, 0, i, 0)),
            pl.BlockSpec((2, RB, 32), lambda i: (0, i, 0)),
            pl.BlockSpec((2, RB, 32), lambda i: (0, i, 0)),
            pl.BlockSpec((1, 32), lambda i: (0, 0)),
            pl.BlockSpec((1, 32), lambda i: (0, 0)),
        ],
        out_specs=pl.BlockSpec((2, RB, 32), lambda i: (0, i, 0)),
        out_shape=jax.ShapeDtypeStruct((2, NP, 32), jnp.float32),
    )(pb, zb, di, b0b, b1b)


def _tc4_body(f_ref, pc_ref, invd_ref, wl1_ref, bl1_ref, wl2_ref, bl2_ref,
              o_ref):
    last = (f_ref[0] + pc_ref[0, 0] + pc_ref[1, 0]) * invd_ref[...]
    h = jnp.maximum(jnp.dot(last, wl1_ref[...],
                            preferred_element_type=jnp.float32)
                    + bl1_ref[...], 0.0)
    o_ref[...] = jnp.maximum(jnp.dot(h, wl2_ref[...],
                                     preferred_element_type=jnp.float32)
                             + bl2_ref[...], 0.0)


def _tc4(f, pc, invd, wl1, bl1, wl2, bl2):
    return pl.pallas_call(
        _tc4_body,
        grid=(NP // RB,),
        in_specs=[
            pl.BlockSpec((1, RB, 32), lambda i: (1, i, 0)),
            pl.BlockSpec((2, 1, RB, 32), lambda i: (0, 0, i, 0)),
            pl.BlockSpec((1, 32), lambda i: (0, 0)),
            pl.BlockSpec((32, 64), lambda i: (0, 0)),
            pl.BlockSpec((1, 64), lambda i: (0, 0)),
            pl.BlockSpec((64, 32), lambda i: (0, 0)),
            pl.BlockSpec((1, 32), lambda i: (0, 0)),
        ],
        out_specs=pl.BlockSpec((RB, 32), lambda i: (i, 0)),
        out_shape=jax.ShapeDtypeStruct((N, 32), jnp.float32),
    )(f, pc, invd, wl1, bl1, wl2, bl2)


def kernel(x0, x1, degrees, edge_index0, edge_index1, layer_edge_index,
           Wg0a, bg0a, Wg0b, bg0b, Wg1a, bg1a, Wg1b, bg1b,
           Wl1, bl1, Wl2, bl2):
    i32 = jnp.int32
    pad_ab = N + (jnp.arange(E_AB - 2 * E, dtype=i32) % (NP - N))
    src_ab = jnp.concatenate(
        [edge_index0[0], edge_index1[0] + NP, pad_ab]).reshape(-1, CH)
    dst_ab = jnp.concatenate(
        [edge_index0[1], edge_index1[1] + NP, pad_ab]).reshape(-1, CH)
    pad_x = N + (jnp.arange(E_X - E, dtype=i32) % (NP - N))
    src_x = jnp.concatenate([layer_edge_index[1], pad_x]).reshape(-1, CH)
    dst_x = jnp.concatenate([layer_edge_index[0], pad_x]).reshape(-1, CH)

    x0p = jnp.pad(x0, ((0, NP - N), (0, 0)))
    x1p = jnp.pad(x1, ((0, NP - N), (0, 0)))
    zeros32 = jnp.zeros((TBL, 32), jnp.float32)
    zeros16 = jnp.zeros((TBL, 16), jnp.float32)
    ones16 = jnp.ones((CH, 16), jnp.float32)
    invd = (1.0 / degrees[1]) * jnp.ones((1, 32), jnp.float32)

    degp = _sc_degree(NSUP_AB)(dst_ab, zeros16, ones16).reshape(2, 2, NP, 16)
    z, di = _tc1(x0p, x1p, Wg0a, Wg1a, degp)
    pa = _sc_gather_scatter(NSUP_AB)(z.reshape(TBL, 32), src_ab, dst_ab,
                                     zeros32)
    zb = _tc2(pa.reshape(2, 2, NP, 32), z, di, Wg0b, Wg1b,
              bg0a.reshape(1, 32), bg1a.reshape(1, 32))
    pb = _sc_gather_scatter(NSUP_AB)(zb.reshape(TBL, 32), src_ab, dst_ab,
                                     zeros32)
    f = _tc3(pb.reshape(2, 2, NP, 32), zb,
             di, bg0b.reshape(1, 32), bg1b.reshape(1, 32))
    pc = _sc_gather_scatter(NSUP_X)(f.reshape(TBL, 32), src_x, dst_x, zeros32)
    out = _tc4(f, pc.reshape(2, 2, NP, 32), invd,
               Wl1, bl1.reshape(1, 64), Wl2, bl2.reshape(1, 32))
    return out


# speedup vs baseline: 36.2811x; 1.0254x over previous
import functools

import jax
import jax.numpy as jnp
from jax import lax
from jax.experimental import pallas as pl
from jax.experimental.pallas import tpu as pltpu
from jax.experimental.pallas import tpu_sc as plsc

N = 10000
E = 320000
NP = 10240
TBL = 2 * NP
NW = 32
CH = 128
K = 16
SUP = CH * K

E_DEG = 655360
E_X = 327680
NSUP_DEG = E_DEG // (NW * SUP)
NSUP_X = E_X // (NW * SUP)

KP = 6
NBAT_AB = 27
E_AB = NW * NBAT_AB * KP * CH

@functools.cache
def _sc_mesh():
    return plsc.VectorSubcoreMesh(core_axis_name="c", subcore_axis_name="s",
                                  num_cores=2, num_subcores=16)


@functools.cache
def _sc_gather_scatter(nsup, nrows):
    rps = nrows // 16

    @functools.partial(
        pl.kernel,
        out_type=jax.ShapeDtypeStruct((2, nrows, 32), jnp.float32),
        mesh=_sc_mesh(),
        compiler_params=pltpu.CompilerParams(use_tc_tiling_on_sc=False),
        scratch_types=[
            pltpu.VMEM((K, CH), jnp.int32),
            pltpu.VMEM((K, CH), jnp.int32),
            pltpu.VMEM((K, CH, 32), jnp.float32),
            pltpu.VMEM_SHARED((nrows, 32), jnp.float32),
            pltpu.SemaphoreType.DMA,
            pltpu.SemaphoreType.DMA,
        ],
    )
    def kern(table_h, src_h, dst_h, zeros_h, out_h,
             idx_s, idx_d, rows, acc, gsem, ssem):
        c = lax.axis_index("c")
        s = lax.axis_index("s")
        wid = s * 2 + c
        pltpu.sync_copy(zeros_h.at[pl.ds(s * rps, rps)],
                        acc.at[pl.ds(s * rps, rps)])
        plsc.subcore_barrier()

        @pl.loop(0, nsup)
        def _(j):
            ch0 = (wid * nsup + j) * K
            pltpu.sync_copy(src_h.at[pl.ds(ch0, K)], idx_s)
            pltpu.sync_copy(dst_h.at[pl.ds(ch0, K)], idx_d)
            gets = [pltpu.async_copy(table_h.at[idx_s.at[t]], rows.at[t], gsem)
                    for t in range(K)]
            for cp in gets:
                cp.wait()
            puts = [pltpu.async_copy(rows.at[t], acc.at[idx_d.at[t]], ssem,
                                     add=True)
                    for t in range(K)]
            for cp in puts:
                cp.wait()

        plsc.subcore_barrier()
        pltpu.sync_copy(acc.at[pl.ds(s * rps, rps)],
                        out_h.at[c].at[pl.ds(s * rps, rps)])

    return kern


@functools.cache
def _sc_gather_scatter_pipe(nbat):
    assert nbat % 3 == 0
    rps = TBL // 16

    @functools.partial(
        pl.kernel,
        out_type=jax.ShapeDtypeStruct((2, TBL, 32), jnp.float32),
        mesh=_sc_mesh(),
        compiler_params=pltpu.CompilerParams(use_tc_tiling_on_sc=False),
        scratch_types=[
            pltpu.VMEM((3, KP, CH), jnp.int32),
            pltpu.VMEM((3, KP, CH), jnp.int32),
            pltpu.VMEM((3, KP, CH, 32), jnp.float32),
            pltpu.VMEM_SHARED((TBL, 32), jnp.float32),
            pltpu.SemaphoreType.DMA,
            pltpu.SemaphoreType.DMA,
            pltpu.SemaphoreType.DMA,
            pltpu.SemaphoreType.DMA,
            pltpu.SemaphoreType.DMA,
            pltpu.SemaphoreType.DMA,
        ],
    )
    def kern(table_h, src_h, dst_h, zeros_h, out_h,
             idx_s, idx_d, rows, acc, g0, g1, g2, s0, s1, s2):
        gsems = (g0, g1, g2)
        ssems = (s0, s1, s2)
        c = lax.axis_index("c")
        s = lax.axis_index("s")
        wid = s * 2 + c
        pltpu.sync_copy(zeros_h.at[pl.ds(s * rps, rps)],
                        acc.at[pl.ds(s * rps, rps)])
        plsc.subcore_barrier()
        base_ch = wid * nbat * KP

        def idx_load(j, b):
            ch0 = base_ch + j * KP
            pltpu.sync_copy(src_h.at[pl.ds(ch0, KP)], idx_s.at[b])
            pltpu.sync_copy(dst_h.at[pl.ds(ch0, KP)], idx_d.at[b])

        def g_fire(b):
            for t in range(KP):
                pltpu.async_copy(table_h.at[idx_s.at[b].at[t]],
                                 rows.at[b].at[t], gsems[b])

        def g_wait(b):
            for t in range(KP):
                pltpu.make_async_copy(table_h.at[idx_s.at[b].at[t]],
                                      rows.at[b].at[t], gsems[b]).wait()

        def s_fire(b):
            for t in range(KP):
                pltpu.async_copy(rows.at[b].at[t],
                                 acc.at[idx_d.at[b].at[t]], ssems[b],
                                 add=True)

        def s_wait(b):
            for t in range(KP):
                pltpu.make_async_copy(rows.at[b].at[t],
                                      acc.at[idx_d.at[b].at[t]],
                                      ssems[b]).wait()

        idx_load(0, 0)
        g_fire(0)
        g_wait(0); s_fire(0); idx_load(1, 1); g_fire(1)
        g_wait(1); s_fire(1); idx_load(2, 2); g_fire(2)

        @pl.loop(0, (nbat - 3) // 3)
        def _(q):
            j = 2 + q * 3
            for off, b, bn in ((0, 2, 0), (1, 0, 1), (2, 1, 2)):
                jj = j + off
                g_wait(b)
                s_fire(b)
                s_wait(bn)
                idx_load(jj + 1, bn)
                g_fire(bn)

        bl = (nbat - 1) % 3
        g_wait(bl)
        s_fire(bl)
        s_wait(0)
        s_wait(1)
        s_wait(2)

        plsc.subcore_barrier()
        pltpu.sync_copy(acc.at[pl.ds(s * rps, rps)],
                        out_h.at[c].at[pl.ds(s * rps, rps)])

    return kern


@functools.cache
def _sc_degree(nsup):
    rps = TBL // 16

    @functools.partial(
        pl.kernel,
        out_type=jax.ShapeDtypeStruct((2, TBL, 16), jnp.float32),
        mesh=_sc_mesh(),
        compiler_params=pltpu.CompilerParams(use_tc_tiling_on_sc=False),
        scratch_types=[
            pltpu.VMEM((K, CH), jnp.int32),
            pltpu.VMEM((CH, 16), jnp.float32),
            pltpu.VMEM_SHARED((TBL, 16), jnp.float32),
            pltpu.SemaphoreType.DMA,
        ],
    )
    def kern(dst_h, zeros_h, ones_h, out_h, idx_d, ones_v, acc, ssem):
        c = lax.axis_index("c")
        s = lax.axis_index("s")
        wid = s * 2 + c
        pltpu.sync_copy(zeros_h.at[pl.ds(s * rps, rps)],
                        acc.at[pl.ds(s * rps, rps)])
        pltpu.sync_copy(ones_h, ones_v)
        plsc.subcore_barrier()

        @pl.loop(0, nsup)
        def _(j):
            ch0 = (wid * nsup + j) * K
            pltpu.sync_copy(dst_h.at[pl.ds(ch0, K)], idx_d)
            puts = [pltpu.async_copy(ones_v, acc.at[idx_d.at[t]], ssem,
                                     add=True)
                    for t in range(K)]
            for cp in puts:
                cp.wait()

        plsc.subcore_barrier()
        pltpu.sync_copy(acc.at[pl.ds(s * rps, rps)],
                        out_h.at[c].at[pl.ds(s * rps, rps)])

    return kern


RB1 = 512
RB = 1024


def _tc1_body(x0_ref, x1_ref, w0_ref, w1_ref, dp_ref, z_ref, di_ref):
    d0 = dp_ref[0, 0] + dp_ref[1, 0]
    d1 = dp_ref[0, 1] + dp_ref[1, 1]
    di0 = lax.rsqrt(d0[:, 0:1] + 1.0)
    di1 = lax.rsqrt(d1[:, 0:1] + 1.0)
    z_ref[0] = jnp.dot(x0_ref[...], w0_ref[...],
                       preferred_element_type=jnp.float32) * di0
    z_ref[1] = jnp.dot(x1_ref[...], w1_ref[...],
                       preferred_element_type=jnp.float32) * di1
    di_ref[0] = jnp.broadcast_to(di0, (RB1, 32))
    di_ref[1] = jnp.broadcast_to(di1, (RB1, 32))


def _tc1(x0p, x1p, w0, w1, degp):
    return pl.pallas_call(
        _tc1_body,
        grid=(NP // RB1,),
        in_specs=[
            pl.BlockSpec((RB1, 128), lambda i: (i, 0)),
            pl.BlockSpec((RB1, 128), lambda i: (i, 0)),
            pl.BlockSpec((128, 32), lambda i: (0, 0)),
            pl.BlockSpec((128, 32), lambda i: (0, 0)),
            pl.BlockSpec((2, 2, RB1, 16), lambda i: (0, 0, i, 0)),
        ],
        out_specs=[
            pl.BlockSpec((2, RB1, 32), lambda i: (0, i, 0)),
            pl.BlockSpec((2, RB1, 32), lambda i: (0, i, 0)),
        ],
        out_shape=[
            jax.ShapeDtypeStruct((2, NP, 32), jnp.float32),
            jax.ShapeDtypeStruct((2, NP, 32), jnp.float32),
        ],
    )(x0p, x1p, w0, w1, degp)


def _tc2_body(pa_ref, z_ref, di_ref, w0_ref, w1_ref, b0_ref, b1_ref, zb_ref):
    a0 = pa_ref[0, 0] + pa_ref[1, 0] + z_ref[0]
    a1 = pa_ref[0, 1] + pa_ref[1, 1] + z_ref[1]
    h0 = jnp.maximum(di_ref[0] * a0 + b0_ref[...], 0.0)
    h1 = jnp.maximum(di_ref[1] * a1 + b1_ref[...], 0.0)
    zb_ref[0] = jnp.dot(h0, w0_ref[...],
                        preferred_element_type=jnp.float32) * di_ref[0]
    zb_ref[1] = jnp.dot(h1, w1_ref[...],
                        preferred_element_type=jnp.float32) * di_ref[1]


def _tc2(pa, z, di, w0b, w1b, b0a, b1a):
    return pl.pallas_call(
        _tc2_body,
        grid=(NP // RB,),
        in_specs=[
            pl.BlockSpec((2, 2, RB, 32), lambda i: (0, 0, i, 0)),
            pl.BlockSpec((2, RB, 32), lambda i: (0, i, 0)),
            pl.BlockSpec((2, RB, 32), lambda i: (0, i, 0)),
            pl.BlockSpec((32, 32), lambda i: (0, 0)),
            pl.BlockSpec((32, 32), lambda i: (0, 0)),
            pl.BlockSpec((1, 32), lambda i: (0, 0)),
            pl.BlockSpec((1, 32), lambda i: (0, 0)),
        ],
        out_specs=pl.BlockSpec((2, RB, 32), lambda i: (0, i, 0)),
        out_shape=jax.ShapeDtypeStruct((2, NP, 32), jnp.float32),
    )(pa, z, di, w0b, w1b, b0a, b1a)


def _tc3_body(pb_ref, zb_ref, di_ref, b0_ref, b1_ref, f_ref):
    f_ref[0] = di_ref[0] * (pb_ref[0, 0] + pb_ref[1, 0] + zb_ref[0]) + b0_ref[...]
    f_ref[1] = di_ref[1] * (pb_ref[0, 1] + pb_ref[1, 1] + zb_ref[1]) + b1_ref[...]


def _tc3(pb, zb, di, b0b, b1b):
    return pl.pallas_call(
        _tc3_body,
        grid=(NP // RB,),
        in_specs=[
            pl.BlockSpec((2, 2, RB, 32), lambda i: (0, 0, i, 0)),
            pl.BlockSpec((2, RB, 32), lambda i: (0, i, 0)),
            pl.BlockSpec((2, RB, 32), lambda i: (0, i, 0)),
            pl.BlockSpec((1, 32), lambda i: (0, 0)),
            pl.BlockSpec((1, 32), lambda i: (0, 0)),
        ],
        out_specs=pl.BlockSpec((2, RB, 32), lambda i: (0, i, 0)),
        out_shape=jax.ShapeDtypeStruct((2, NP, 32), jnp.float32),
    )(pb, zb, di, b0b, b1b)


def _tc4_body(f_ref, pc_ref, invd_ref, wl1_ref, bl1_ref, wl2_ref, bl2_ref,
              o_ref):
    last = (f_ref[0] + pc_ref[0] + pc_ref[1]) * invd_ref[...]
    h = jnp.maximum(jnp.dot(last, wl1_ref[...],
                            preferred_element_type=jnp.float32)
                    + bl1_ref[...], 0.0)
    o_ref[...] = jnp.maximum(jnp.dot(h, wl2_ref[...],
                                     preferred_element_type=jnp.float32)
                             + bl2_ref[...], 0.0)


def _tc4(f, pc, invd, wl1, bl1, wl2, bl2):
    return pl.pallas_call(
        _tc4_body,
        grid=(NP // RB,),
        in_specs=[
            pl.BlockSpec((1, RB, 32), lambda i: (1, i, 0)),
            pl.BlockSpec((2, RB, 32), lambda i: (0, i, 0)),
            pl.BlockSpec((1, 32), lambda i: (0, 0)),
            pl.BlockSpec((32, 64), lambda i: (0, 0)),
            pl.BlockSpec((1, 64), lambda i: (0, 0)),
            pl.BlockSpec((64, 32), lambda i: (0, 0)),
            pl.BlockSpec((1, 32), lambda i: (0, 0)),
        ],
        out_specs=pl.BlockSpec((RB, 32), lambda i: (i, 0)),
        out_shape=jax.ShapeDtypeStruct((N, 32), jnp.float32),
    )(f, pc, invd, wl1, bl1, wl2, bl2)


def kernel(x0, x1, degrees, edge_index0, edge_index1, layer_edge_index,
           Wg0a, bg0a, Wg0b, bg0b, Wg1a, bg1a, Wg1b, bg1b,
           Wl1, bl1, Wl2, bl2):
    i32 = jnp.int32
    pad_ab = N + (jnp.arange(E_AB - 2 * E, dtype=i32) % (NP - N))
    src_ab = jnp.concatenate(
        [edge_index0[0], edge_index1[0] + NP, pad_ab]).reshape(-1, CH)
    dst_ab = jnp.concatenate(
        [edge_index0[1], edge_index1[1] + NP, pad_ab]).reshape(-1, CH)
    pad_deg = N + (jnp.arange(E_DEG - 2 * E, dtype=i32) % (NP - N))
    dst_deg = jnp.concatenate(
        [edge_index0[1], edge_index1[1] + NP, pad_deg]).reshape(-1, CH)
    pad_x = N + (jnp.arange(E_X - E, dtype=i32) % (NP - N))
    src_x = jnp.concatenate([layer_edge_index[1], pad_x]).reshape(-1, CH)
    dst_x = jnp.concatenate([layer_edge_index[0], pad_x]).reshape(-1, CH)

    x0p = jnp.pad(x0, ((0, NP - N), (0, 0)))
    x1p = jnp.pad(x1, ((0, NP - N), (0, 0)))
    zeros32 = jnp.zeros((TBL, 32), jnp.float32)
    zeros16 = jnp.zeros((TBL, 16), jnp.float32)
    ones16 = jnp.ones((CH, 16), jnp.float32)
    invd = (1.0 / degrees[1]) * jnp.ones((1, 32), jnp.float32)

    degp = _sc_degree(NSUP_DEG)(dst_deg, zeros16, ones16).reshape(2, 2, NP, 16)
    z, di = _tc1(x0p, x1p, Wg0a, Wg1a, degp)
    pa = _sc_gather_scatter_pipe(NBAT_AB)(z.reshape(TBL, 32), src_ab, dst_ab,
                                          zeros32)
    zb = _tc2(pa.reshape(2, 2, NP, 32), z, di, Wg0b, Wg1b,
              bg0a.reshape(1, 32), bg1a.reshape(1, 32))
    pb = _sc_gather_scatter_pipe(NBAT_AB)(zb.reshape(TBL, 32), src_ab, dst_ab,
                                          zeros32)
    f = _tc3(pb.reshape(2, 2, NP, 32), zb,
             di, bg0b.reshape(1, 32), bg1b.reshape(1, 32))
    pc = _sc_gather_scatter(NSUP_X, NP)(f.reshape(TBL, 32), src_x, dst_x,
                                        zeros32[:NP])
    out = _tc4(f, pc, invd,
               Wl1, bl1.reshape(1, 64), Wl2, bl2.reshape(1, 32))
    return out


# speedup vs baseline: 36.9110x; 1.0174x over previous
import functools

import jax
import jax.numpy as jnp
from jax import lax
from jax.experimental import pallas as pl
from jax.experimental.pallas import tpu as pltpu
from jax.experimental.pallas import tpu_sc as plsc

N = 10000
E = 320000
NP = 10240
TBL = 2 * NP
NW = 32
CH = 128
K = 16
SUP = CH * K

E_DEG = 655360
E_X = 327680
NSUP_DEG = E_DEG // (NW * SUP)
NSUP_X = E_X // (NW * SUP)

KP = 6
NBAT_AB = 27
E_AB = NW * NBAT_AB * KP * CH

@functools.cache
def _sc_mesh():
    return plsc.VectorSubcoreMesh(core_axis_name="c", subcore_axis_name="s",
                                  num_cores=2, num_subcores=16)


@functools.cache
def _sc_gather_scatter(nsup, nrows):
    rps = nrows // 16

    @functools.partial(
        pl.kernel,
        out_type=jax.ShapeDtypeStruct((2, nrows, 32), jnp.float32),
        mesh=_sc_mesh(),
        compiler_params=pltpu.CompilerParams(use_tc_tiling_on_sc=False),
        scratch_types=[
            pltpu.VMEM((K, CH), jnp.int32),
            pltpu.VMEM((K, CH), jnp.int32),
            pltpu.VMEM((K, CH, 32), jnp.float32),
            pltpu.VMEM_SHARED((nrows, 32), jnp.float32),
            pltpu.SemaphoreType.DMA,
            pltpu.SemaphoreType.DMA,
        ],
    )
    def kern(table_h, src_h, dst_h, zeros_h, out_h,
             idx_s, idx_d, rows, acc, gsem, ssem):
        c = lax.axis_index("c")
        s = lax.axis_index("s")
        wid = s * 2 + c
        pltpu.sync_copy(zeros_h.at[pl.ds(s * rps, rps)],
                        acc.at[pl.ds(s * rps, rps)])
        plsc.subcore_barrier()

        @pl.loop(0, nsup)
        def _(j):
            ch0 = (wid * nsup + j) * K
            pltpu.sync_copy(src_h.at[pl.ds(ch0, K)], idx_s)
            pltpu.sync_copy(dst_h.at[pl.ds(ch0, K)], idx_d)
            gets = [pltpu.async_copy(table_h.at[idx_s.at[t]], rows.at[t], gsem)
                    for t in range(K)]
            for cp in gets:
                cp.wait()
            puts = [pltpu.async_copy(rows.at[t], acc.at[idx_d.at[t]], ssem,
                                     add=True)
                    for t in range(K)]
            for cp in puts:
                cp.wait()

        plsc.subcore_barrier()
        pltpu.sync_copy(acc.at[pl.ds(s * rps, rps)],
                        out_h.at[c].at[pl.ds(s * rps, rps)])

    return kern


@functools.cache
def _sc_gather_scatter_pipe(nbat):
    assert nbat % 3 == 0
    rps = TBL // 16

    @functools.partial(
        pl.kernel,
        out_type=jax.ShapeDtypeStruct((2, TBL, 32), jnp.float32),
        mesh=_sc_mesh(),
        compiler_params=pltpu.CompilerParams(use_tc_tiling_on_sc=False),
        scratch_types=[
            pltpu.VMEM((3, KP, CH), jnp.int32),
            pltpu.VMEM((3, KP, CH), jnp.int32),
            pltpu.VMEM((3, KP, CH, 32), jnp.float32),
            pltpu.VMEM_SHARED((TBL, 32), jnp.float32),
            pltpu.SemaphoreType.DMA,
            pltpu.SemaphoreType.DMA,
            pltpu.SemaphoreType.DMA,
            pltpu.SemaphoreType.DMA,
            pltpu.SemaphoreType.DMA,
            pltpu.SemaphoreType.DMA,
        ],
    )
    def kern(table_h, src_h, dst_h, zeros_h, out_h,
             idx_s, idx_d, rows, acc, g0, g1, g2, s0, s1, s2):
        gsems = (g0, g1, g2)
        ssems = (s0, s1, s2)
        c = lax.axis_index("c")
        s = lax.axis_index("s")
        wid = s * 2 + c
        pltpu.sync_copy(zeros_h.at[pl.ds(s * rps, rps)],
                        acc.at[pl.ds(s * rps, rps)])
        plsc.subcore_barrier()
        base_ch = wid * nbat * KP

        def idx_load(j, b):
            ch0 = base_ch + j * KP
            pltpu.sync_copy(src_h.at[pl.ds(ch0, KP)], idx_s.at[b])
            pltpu.sync_copy(dst_h.at[pl.ds(ch0, KP)], idx_d.at[b])

        def g_fire(b):
            for t in range(KP):
                pltpu.async_copy(table_h.at[idx_s.at[b].at[t]],
                                 rows.at[b].at[t], gsems[b])

        def g_wait(b):
            for t in range(KP):
                pltpu.make_async_copy(table_h.at[idx_s.at[b].at[t]],
                                      rows.at[b].at[t], gsems[b]).wait()

        def s_fire(b):
            for t in range(KP):
                pltpu.async_copy(rows.at[b].at[t],
                                 acc.at[idx_d.at[b].at[t]], ssems[b],
                                 add=True)

        def s_wait(b):
            for t in range(KP):
                pltpu.make_async_copy(rows.at[b].at[t],
                                      acc.at[idx_d.at[b].at[t]],
                                      ssems[b]).wait()

        idx_load(0, 0)
        g_fire(0)
        g_wait(0); s_fire(0); idx_load(1, 1); g_fire(1)
        g_wait(1); s_fire(1); idx_load(2, 2); g_fire(2)

        @pl.loop(0, (nbat - 3) // 3)
        def _(q):
            j = 2 + q * 3
            for off, b, bn in ((0, 2, 0), (1, 0, 1), (2, 1, 2)):
                jj = j + off
                g_wait(b)
                s_fire(b)
                s_wait(bn)
                idx_load(jj + 1, bn)
                g_fire(bn)

        bl = (nbat - 1) % 3
        g_wait(bl)
        s_fire(bl)
        s_wait(0)
        s_wait(1)
        s_wait(2)

        plsc.subcore_barrier()
        pltpu.sync_copy(acc.at[pl.ds(s * rps, rps)],
                        out_h.at[c].at[pl.ds(s * rps, rps)])

    return kern


@functools.cache
def _sc_degree(nsup):
    rps = TBL // 16

    @functools.partial(
        pl.kernel,
        out_type=jax.ShapeDtypeStruct((2, TBL, 16), jnp.float32),
        mesh=_sc_mesh(),
        compiler_params=pltpu.CompilerParams(use_tc_tiling_on_sc=False),
        scratch_types=[
            pltpu.VMEM((K, CH), jnp.int32),
            pltpu.VMEM((CH, 16), jnp.float32),
            pltpu.VMEM_SHARED((TBL, 16), jnp.float32),
            pltpu.SemaphoreType.DMA,
        ],
    )
    def kern(dst_h, zeros_h, ones_h, out_h, idx_d, ones_v, acc, ssem):
        c = lax.axis_index("c")
        s = lax.axis_index("s")
        wid = s * 2 + c
        pltpu.sync_copy(zeros_h.at[pl.ds(s * rps, rps)],
                        acc.at[pl.ds(s * rps, rps)])
        pltpu.sync_copy(ones_h, ones_v)
        plsc.subcore_barrier()

        @pl.loop(0, nsup)
        def _(j):
            ch0 = (wid * nsup + j) * K
            pltpu.sync_copy(dst_h.at[pl.ds(ch0, K)], idx_d)
            puts = [pltpu.async_copy(ones_v, acc.at[idx_d.at[t]], ssem,
                                     add=True)
                    for t in range(K)]
            for cp in puts:
                cp.wait()

        plsc.subcore_barrier()
        pltpu.sync_copy(acc.at[pl.ds(s * rps, rps)],
                        out_h.at[c].at[pl.ds(s * rps, rps)])

    return kern


RB = 2048
NBN = NP // RB


def _tc1_body(x_ref, w_ref, dp_ref, z_ref, di_ref):
    di = lax.rsqrt(dp_ref[0][:, 0:1] + dp_ref[1][:, 0:1] + 1.0)
    z_ref[...] = jnp.dot(x_ref[...], w_ref[0],
                         preferred_element_type=jnp.float32) * di
    di_ref[...] = jnp.broadcast_to(di, (RB, 32))


def _tc1(xcat, wcat, degp):
    return pl.pallas_call(
        _tc1_body,
        grid=(TBL // RB,),
        in_specs=[
            pl.BlockSpec((RB, 128), lambda i: (i, 0)),
            pl.BlockSpec((1, 128, 32), lambda i: (i // NBN, 0, 0)),
            pl.BlockSpec((2, RB, 16), lambda i: (0, i, 0)),
        ],
        out_specs=[
            pl.BlockSpec((RB, 32), lambda i: (i, 0)),
            pl.BlockSpec((RB, 32), lambda i: (i, 0)),
        ],
        out_shape=[
            jax.ShapeDtypeStruct((TBL, 32), jnp.float32),
            jax.ShapeDtypeStruct((TBL, 32), jnp.float32),
        ],
    )(xcat, wcat, degp)


def _tc2_body(pa_ref, z_ref, di_ref, w_ref, b_ref, zb_ref):
    a = pa_ref[0] + pa_ref[1] + z_ref[...]
    h = jnp.maximum(di_ref[...] * a + b_ref[0], 0.0)
    zb_ref[...] = jnp.dot(h, w_ref[0],
                          preferred_element_type=jnp.float32) * di_ref[...]


def _tc2(pa, z, di, wbcat, bacat):
    return pl.pallas_call(
        _tc2_body,
        grid=(TBL // RB,),
        in_specs=[
            pl.BlockSpec((2, RB, 32), lambda i: (0, i, 0)),
            pl.BlockSpec((RB, 32), lambda i: (i, 0)),
            pl.BlockSpec((RB, 32), lambda i: (i, 0)),
            pl.BlockSpec((1, 32, 32), lambda i: (i // NBN, 0, 0)),
            pl.BlockSpec((1, 1, 32), lambda i: (i // NBN, 0, 0)),
        ],
        out_specs=pl.BlockSpec((RB, 32), lambda i: (i, 0)),
        out_shape=jax.ShapeDtypeStruct((TBL, 32), jnp.float32),
    )(pa, z, di, wbcat, bacat)


def _tc3_body(pb_ref, zb_ref, di_ref, b_ref, f_ref):
    f_ref[...] = (di_ref[...] * (pb_ref[0] + pb_ref[1] + zb_ref[...])
                  + b_ref[0])


def _tc3(pb, zb, di, bbcat):
    return pl.pallas_call(
        _tc3_body,
        grid=(TBL // RB,),
        in_specs=[
            pl.BlockSpec((2, RB, 32), lambda i: (0, i, 0)),
            pl.BlockSpec((RB, 32), lambda i: (i, 0)),
            pl.BlockSpec((RB, 32), lambda i: (i, 0)),
            pl.BlockSpec((1, 1, 32), lambda i: (i // NBN, 0, 0)),
        ],
        out_specs=pl.BlockSpec((RB, 32), lambda i: (i, 0)),
        out_shape=jax.ShapeDtypeStruct((TBL, 32), jnp.float32),
    )(pb, zb, di, bbcat)


def _tc4_body(f_ref, pc_ref, invd_ref, wl1_ref, bl1_ref, wl2_ref, bl2_ref,
              o_ref):
    last = (f_ref[...] + pc_ref[0] + pc_ref[1]) * invd_ref[...]
    h = jnp.maximum(jnp.dot(last, wl1_ref[...],
                            preferred_element_type=jnp.float32)
                    + bl1_ref[...], 0.0)
    o_ref[...] = jnp.maximum(jnp.dot(h, wl2_ref[...],
                                     preferred_element_type=jnp.float32)
                             + bl2_ref[...], 0.0)


def _tc4(f, pc, invd, wl1, bl1, wl2, bl2):
    return pl.pallas_call(
        _tc4_body,
        grid=(NP // RB,),
        in_specs=[
            pl.BlockSpec((RB, 32), lambda i: (i + NBN, 0)),
            pl.BlockSpec((2, RB, 32), lambda i: (0, i, 0)),
            pl.BlockSpec((1, 32), lambda i: (0, 0)),
            pl.BlockSpec((32, 64), lambda i: (0, 0)),
            pl.BlockSpec((1, 64), lambda i: (0, 0)),
            pl.BlockSpec((64, 32), lambda i: (0, 0)),
            pl.BlockSpec((1, 32), lambda i: (0, 0)),
        ],
        out_specs=pl.BlockSpec((RB, 32), lambda i: (i, 0)),
        out_shape=jax.ShapeDtypeStruct((N, 32), jnp.float32),
    )(f, pc, invd, wl1, bl1, wl2, bl2)


def kernel(x0, x1, degrees, edge_index0, edge_index1, layer_edge_index,
           Wg0a, bg0a, Wg0b, bg0b, Wg1a, bg1a, Wg1b, bg1b,
           Wl1, bl1, Wl2, bl2):
    i32 = jnp.int32
    pad_ab = N + (jnp.arange(E_AB - 2 * E, dtype=i32) % (NP - N))
    src_ab = jnp.concatenate(
        [edge_index0[0], edge_index1[0] + NP, pad_ab]).reshape(-1, CH)
    dst_ab = jnp.concatenate(
        [edge_index0[1], edge_index1[1] + NP, pad_ab]).reshape(-1, CH)
    pad_deg = N + (jnp.arange(E_DEG - 2 * E, dtype=i32) % (NP - N))
    dst_deg = jnp.concatenate(
        [edge_index0[1], edge_index1[1] + NP, pad_deg]).reshape(-1, CH)
    pad_x = N + (jnp.arange(E_X - E, dtype=i32) % (NP - N))
    src_x = jnp.concatenate([layer_edge_index[1], pad_x]).reshape(-1, CH)
    dst_x = jnp.concatenate([layer_edge_index[0], pad_x]).reshape(-1, CH)

    xcat = jnp.concatenate([jnp.pad(x0, ((0, NP - N), (0, 0))),
                            jnp.pad(x1, ((0, NP - N), (0, 0)))])
    wacat = jnp.stack([Wg0a, Wg1a])
    wbcat = jnp.stack([Wg0b, Wg1b])
    bacat = jnp.stack([bg0a, bg1a]).reshape(2, 1, 32)
    bbcat = jnp.stack([bg0b, bg1b]).reshape(2, 1, 32)
    zeros32 = jnp.zeros((TBL, 32), jnp.float32)
    zeros16 = jnp.zeros((TBL, 16), jnp.float32)
    ones16 = jnp.ones((CH, 16), jnp.float32)
    invd = (1.0 / degrees[1]) * jnp.ones((1, 32), jnp.float32)

    degp = _sc_degree(NSUP_DEG)(dst_deg, zeros16, ones16)
    z, di = _tc1(xcat, wacat, degp)
    pa = _sc_gather_scatter_pipe(NBAT_AB)(z, src_ab, dst_ab, zeros32)
    zb = _tc2(pa, z, di, wbcat, bacat)
    pb = _sc_gather_scatter_pipe(NBAT_AB)(zb, src_ab, dst_ab, zeros32)
    f = _tc3(pb, zb, di, bbcat)
    pc = _sc_gather_scatter(NSUP_X, NP)(f, src_x, dst_x, zeros32[:NP])
    out = _tc4(f, pc, invd,
               Wl1, bl1.reshape(1, 64), Wl2, bl2.reshape(1, 32))
    return out
